# baseline (device time: 40653 ns/iter reference)
import jax
import jax.numpy as jnp
from jax import lax
from jax.experimental import pallas as pl
from jax.experimental.pallas import tpu as pltpu

NZ = 4


def kernel(x):
    m, n = x.shape
    m2 = m // 2
    ch = m2 // NZ

    def body(x_hbm, out_ref, xv, xh, rs_buf, copy_sem, rs_send, rs_recv,
             aga_send, aga_recv, agb_send, agb_recv, xg_send, xg_recv):
        my_x = lax.axis_index("x")
        my_y = lax.axis_index("y")
        my_z = lax.axis_index("z")
        base = my_x * m2
        obase = (1 - my_x) * m2
        own = base + my_z * ch
        pown = obase + my_z * ch

        cp = pltpu.make_async_copy(
            x_hbm.at[pl.ds(base, m2), :], xv, copy_sem
        )
        cp.start()

        bsem = pltpu.get_barrier_semaphore()
        for k in range(1, NZ):
            pl.semaphore_signal(
                bsem, inc=1,
                device_id=(my_x, my_y, (my_z + k) % NZ),
                device_id_type=pl.DeviceIdType.MESH,
            )
        pl.semaphore_signal(
            bsem, inc=1,
            device_id=(1 - my_x, my_y, my_z),
            device_id_type=pl.DeviceIdType.MESH,
        )
        pl.semaphore_wait(bsem, NZ)

        cp.wait()
        xh[:, :] = xv[:, :].astype(jnp.bfloat16)

        rs = []
        for k in range(1, NZ):
            dst = (my_z + k) % NZ
            rdma = pltpu.make_async_remote_copy(
                src_ref=xh.at[pl.ds(dst * ch, ch), :],
                dst_ref=rs_buf.at[k - 1],
                send_sem=rs_send.at[k - 1],
                recv_sem=rs_recv.at[k - 1],
                device_id=(my_x, my_y, dst),
                device_id_type=pl.DeviceIdType.MESH,
            )
            rdma.start()
            rs.append(rdma)
        for rdma in rs:
            rdma.wait_recv()

        acc = xh[pl.ds(my_z * ch, ch), :].astype(jnp.float32)
        for k in range(1, NZ):
            acc += rs_buf[k - 1, :, :].astype(jnp.float32)
        out_ref[pl.ds(own, ch), :] = acc.astype(jnp.bfloat16)

        g0 = pltpu.make_async_remote_copy(
            src_ref=out_ref.at[pl.ds(own, ch), :],
            dst_ref=out_ref.at[pl.ds(own, ch), :],
            send_sem=xg_send.at[0],
            recv_sem=xg_recv.at[0],
            device_id=(1 - my_x, my_y, my_z),
            device_id_type=pl.DeviceIdType.MESH,
        )
        g0.start()

        aga = []
        for k in range(1, NZ):
            dst = (my_z + k) % NZ
            rdma = pltpu.make_async_remote_copy(
                src_ref=out_ref.at[pl.ds(own, ch), :],
                dst_ref=out_ref.at[pl.ds(own, ch), :],
                send_sem=aga_send.at[k - 1],
                recv_sem=aga_recv.at[k - 1],
                device_id=(my_x, my_y, dst),
                device_id_type=pl.DeviceIdType.MESH,
            )
            rdma.start()
            aga.append(rdma)

        g0.wait_recv()
        agb = []
        for k in range(1, NZ):
            dst = (my_z + k) % NZ
            rdma = pltpu.make_async_remote_copy(
                src_ref=out_ref.at[pl.ds(pown, ch), :],
                dst_ref=out_ref.at[pl.ds(pown, ch), :],
                send_sem=agb_send.at[k - 1],
                recv_sem=agb_recv.at[k - 1],
                device_id=(my_x, my_y, dst),
                device_id_type=pl.DeviceIdType.MESH,
            )
            rdma.start()
            agb.append(rdma)

        for rdma in aga:
            rdma.wait_recv()
        for rdma in agb:
            rdma.wait_recv()
        for rdma in rs:
            rdma.wait_send()
        g0.wait_send()
        for rdma in aga:
            rdma.wait_send()
        for rdma in agb:
            rdma.wait_send()

    return pl.pallas_call(
        body,
        out_shape=jax.ShapeDtypeStruct((m, n), jnp.bfloat16),
        in_specs=[pl.BlockSpec(memory_space=pl.ANY)],
        out_specs=pl.BlockSpec(memory_space=pltpu.VMEM),
        scratch_shapes=[
            pltpu.VMEM((m2, n), jnp.float32),
            pltpu.VMEM((m2, n), jnp.bfloat16),
            pltpu.VMEM((NZ - 1, ch, n), jnp.bfloat16),
            pltpu.SemaphoreType.DMA,
            pltpu.SemaphoreType.DMA((NZ - 1,)),
            pltpu.SemaphoreType.DMA((NZ - 1,)),
            pltpu.SemaphoreType.DMA((NZ - 1,)),
            pltpu.SemaphoreType.DMA((NZ - 1,)),
            pltpu.SemaphoreType.DMA((NZ - 1,)),
            pltpu.SemaphoreType.DMA((NZ - 1,)),
            pltpu.SemaphoreType.DMA((1,)),
            pltpu.SemaphoreType.DMA((1,)),
        ],
        compiler_params=pltpu.CompilerParams(collective_id=0),
    )(x)


# device time: 35924 ns/iter; 1.1316x vs baseline; 1.1316x over previous
import jax
import jax.numpy as jnp
from jax import lax
from jax.experimental import pallas as pl
from jax.experimental.pallas import tpu as pltpu

NZ = 4
S = 2


def kernel(x):
    m, n = x.shape
    m2 = m // 2
    ch = m2 // NZ
    hh = ch // S

    def body(x_hbm, out_ref, xv, xh, rs_buf, copy_sem, rs_send, rs_recv,
             ag_send, ag_recv, xg_send, xg_recv):
        my_x = lax.axis_index("x")
        my_y = lax.axis_index("y")
        my_z = lax.axis_index("z")
        base = my_x * m2
        own = base + my_z * ch

        cp = pltpu.make_async_copy(
            x_hbm.at[pl.ds(base, m2), :], xv, copy_sem
        )
        cp.start()

        bsem = pltpu.get_barrier_semaphore()
        for k in range(1, NZ):
            pl.semaphore_signal(
                bsem, inc=1,
                device_id=(my_x, my_y, (my_z + k) % NZ),
                device_id_type=pl.DeviceIdType.MESH,
            )
        pl.semaphore_signal(
            bsem, inc=1,
            device_id=(1 - my_x, my_y, my_z),
            device_id_type=pl.DeviceIdType.MESH,
        )
        pl.semaphore_wait(bsem, NZ)

        cp.wait()
        xh[:, :] = xv[:, :].astype(jnp.bfloat16)

        rs = []
        for s in range(S):
            for k in range(1, NZ):
                dst = (my_z + k) % NZ
                slot = s * (NZ - 1) + (k - 1)
                rdma = pltpu.make_async_remote_copy(
                    src_ref=xh.at[pl.ds(dst * ch + s * hh, hh), :],
                    dst_ref=rs_buf.at[slot],
                    send_sem=rs_send.at[slot],
                    recv_sem=rs_recv.at[slot],
                    device_id=(my_x, my_y, dst),
                    device_id_type=pl.DeviceIdType.MESH,
                )
                rdma.start()
                rs.append(rdma)

        ag = []
        xg = []
        for s in range(S):
            for k in range(1, NZ):
                rs[s * (NZ - 1) + (k - 1)].wait_recv()
            acc = xh[pl.ds(my_z * ch + s * hh, hh), :].astype(jnp.float32)
            for k in range(1, NZ):
                slot = s * (NZ - 1) + (k - 1)
                acc += rs_buf[slot, :, :].astype(jnp.float32)
            rows = pl.ds(own + s * hh, hh)
            out_ref[rows, :] = acc.astype(jnp.bfloat16)

            g = pltpu.make_async_remote_copy(
                src_ref=out_ref.at[rows, :],
                dst_ref=out_ref.at[rows, :],
                send_sem=xg_send.at[s],
                recv_sem=xg_recv.at[s],
                device_id=(1 - my_x, my_y, my_z),
                device_id_type=pl.DeviceIdType.MESH,
            )
            g.start()
            xg.append(g)

            for k in range(1, NZ):
                dst = (my_z + k) % NZ
                slot = s * (NZ - 1) + (k - 1)
                rdma = pltpu.make_async_remote_copy(
                    src_ref=out_ref.at[rows, :],
                    dst_ref=out_ref.at[rows, :],
                    send_sem=ag_send.at[slot],
                    recv_sem=ag_recv.at[slot],
                    device_id=(my_x, my_y, dst),
                    device_id_type=pl.DeviceIdType.MESH,
                )
                rdma.start()
                ag.append(rdma)

        for s in range(S):
            for k in range(1, NZ):
                slot = s * (NZ - 1) + (k - 1)
                ag[slot].wait_recv()
                src_z = (my_z - k) % NZ
                rows = pl.ds(base + src_z * ch + s * hh, hh)
                g = pltpu.make_async_remote_copy(
                    src_ref=out_ref.at[rows, :],
                    dst_ref=out_ref.at[rows, :],
                    send_sem=xg_send.at[S + slot],
                    recv_sem=xg_recv.at[S + slot],
                    device_id=(1 - my_x, my_y, my_z),
                    device_id_type=pl.DeviceIdType.MESH,
                )
                g.start()
                xg.append(g)

        for g in xg:
            g.wait()
        for rdma in rs:
            rdma.wait_send()
        for rdma in ag:
            rdma.wait_send()

    n_slots = S * (NZ - 1)
    n_xg = S + n_slots

    return pl.pallas_call(
        body,
        out_shape=jax.ShapeDtypeStruct((m, n), jnp.bfloat16),
        in_specs=[pl.BlockSpec(memory_space=pl.ANY)],
        out_specs=pl.BlockSpec(memory_space=pltpu.VMEM),
        scratch_shapes=[
            pltpu.VMEM((m2, n), jnp.float32),
            pltpu.VMEM((m2, n), jnp.bfloat16),
            pltpu.VMEM((n_slots, hh, n), jnp.bfloat16),
            pltpu.SemaphoreType.DMA,
            pltpu.SemaphoreType.DMA((n_slots,)),
            pltpu.SemaphoreType.DMA((n_slots,)),
            pltpu.SemaphoreType.DMA((n_slots,)),
            pltpu.SemaphoreType.DMA((n_slots,)),
            pltpu.SemaphoreType.DMA((n_xg,)),
            pltpu.SemaphoreType.DMA((n_xg,)),
        ],
        compiler_params=pltpu.CompilerParams(collective_id=0),
    )(x)


# device time: 34957 ns/iter; 1.1629x vs baseline; 1.0277x over previous
import jax
import jax.numpy as jnp
from jax import lax
from jax.experimental import pallas as pl
from jax.experimental.pallas import tpu as pltpu

NZ = 4
S = 2


def kernel(x):
    m, n = x.shape
    m2 = m // 2
    ch = m2 // NZ
    hh = ch // S

    def body(x_hbm, out_ref, xv, xh, rs_buf, copy_sem, rs_send, rs_recv,
             ag_send, ag_recv, yf_send, yf_recv, xg_send, xg_recv):
        my_x = lax.axis_index("x")
        my_y = lax.axis_index("y")
        my_z = lax.axis_index("z")
        base = my_x * m2
        own = base + my_z * ch
        buddy_y = my_y + 1 - 2 * (my_y % 2)
        par = my_y % 2

        cp = pltpu.make_async_copy(
            x_hbm.at[pl.ds(base, m2), :], xv, copy_sem
        )
        cp.start()

        bsem = pltpu.get_barrier_semaphore()
        for k in range(1, NZ):
            pl.semaphore_signal(
                bsem, inc=1,
                device_id=(my_x, my_y, (my_z + k) % NZ),
                device_id_type=pl.DeviceIdType.MESH,
            )
        pl.semaphore_signal(
            bsem, inc=1,
            device_id=(1 - my_x, my_y, my_z),
            device_id_type=pl.DeviceIdType.MESH,
        )
        pl.semaphore_signal(
            bsem, inc=1,
            device_id=(my_x, buddy_y, my_z),
            device_id_type=pl.DeviceIdType.MESH,
        )
        pl.semaphore_wait(bsem, NZ + 1)

        cp.wait()
        xh[:, :] = xv[:, :].astype(jnp.bfloat16)

        rs = []
        for s in range(S):
            for k in range(1, NZ):
                dst = (my_z + k) % NZ
                slot = s * (NZ - 1) + (k - 1)
                rdma = pltpu.make_async_remote_copy(
                    src_ref=xh.at[pl.ds(dst * ch + s * hh, hh), :],
                    dst_ref=rs_buf.at[slot],
                    send_sem=rs_send.at[slot],
                    recv_sem=rs_recv.at[slot],
                    device_id=(my_x, my_y, dst),
                    device_id_type=pl.DeviceIdType.MESH,
                )
                rdma.start()
                rs.append(rdma)

        ag = []
        xg = []
        for s in range(S):
            for k in range(1, NZ):
                rs[s * (NZ - 1) + (k - 1)].wait_recv()
            acc = xh[pl.ds(my_z * ch + s * hh, hh), :].astype(jnp.float32)
            for k in range(1, NZ):
                slot = s * (NZ - 1) + (k - 1)
                acc += rs_buf[slot, :, :].astype(jnp.float32)
            rows = pl.ds(own + s * hh, hh)
            out_ref[rows, :] = acc.astype(jnp.bfloat16)

            g = pltpu.make_async_remote_copy(
                src_ref=out_ref.at[rows, :],
                dst_ref=out_ref.at[rows, :],
                send_sem=xg_send.at[s],
                recv_sem=xg_recv.at[s],
                device_id=(1 - my_x, my_y, my_z),
                device_id_type=pl.DeviceIdType.MESH,
            )
            g.start()
            xg.append(g)

            for k in range(1, NZ):
                dst = (my_z + k) % NZ
                slot = s * (NZ - 1) + (k - 1)
                rdma = pltpu.make_async_remote_copy(
                    src_ref=out_ref.at[rows, :],
                    dst_ref=out_ref.at[rows, :],
                    send_sem=ag_send.at[slot],
                    recv_sem=ag_recv.at[slot],
                    device_id=(my_x, my_y, dst),
                    device_id_type=pl.DeviceIdType.MESH,
                )
                ag.append(rdma)

                @pl.when(my_z % 2 == par)
                def _(rdma=rdma):
                    rdma.start()

        yf = []
        for s in range(S):
            for k in range(1, NZ):
                slot = s * (NZ - 1) + (k - 1)
                c = (my_z - k) % NZ
                rows = pl.ds(base + c * ch + s * hh, hh)
                in_par = c % 2 == par

                @pl.when(in_par)
                def _(slot=slot):
                    ag[slot].wait_recv()

                gx = pltpu.make_async_remote_copy(
                    src_ref=out_ref.at[rows, :],
                    dst_ref=out_ref.at[rows, :],
                    send_sem=xg_send.at[S + slot],
                    recv_sem=xg_recv.at[S + slot],
                    device_id=(1 - my_x, my_y, my_z),
                    device_id_type=pl.DeviceIdType.MESH,
                )
                gy = pltpu.make_async_remote_copy(
                    src_ref=out_ref.at[rows, :],
                    dst_ref=out_ref.at[rows, :],
                    send_sem=yf_send.at[slot],
                    recv_sem=yf_recv.at[slot],
                    device_id=(my_x, buddy_y, my_z),
                    device_id_type=pl.DeviceIdType.MESH,
                )
                xg.append(gx)
                yf.append(gy)

                @pl.when(in_par)
                def _(gx=gx, gy=gy):
                    gx.start()
                    gy.start()

        for s in range(S):
            for k in range(1, NZ):
                slot = s * (NZ - 1) + (k - 1)
                c = (my_z - k) % NZ
                rows = pl.ds(base + c * ch + s * hh, hh)
                off_par = c % 2 != par

                @pl.when(off_par)
                def _(slot=slot):
                    yf[slot].wait_recv()

                gx = xg[S + slot]

                @pl.when(off_par)
                def _(gx=gx):
                    gx.start()

        for g in xg:
            g.wait_recv()
        for g in xg[:S]:
            g.wait_send()
        for s in range(S):
            for k in range(1, NZ):
                slot = s * (NZ - 1) + (k - 1)
                c = (my_z - k) % NZ

                @pl.when(c % 2 == par)
                def _(slot=slot):
                    xg[S + slot].wait_send()
                    yf[slot].wait_send()

                @pl.when(c % 2 != par)
                def _(slot=slot):
                    xg[S + slot].wait_send()

                @pl.when(my_z % 2 == par)
                def _(slot=slot):
                    ag[slot].wait_send()
        for rdma in rs:
            rdma.wait_send()

    n_slots = S * (NZ - 1)
    n_xg = S + n_slots

    return pl.pallas_call(
        body,
        out_shape=jax.ShapeDtypeStruct((m, n), jnp.bfloat16),
        in_specs=[pl.BlockSpec(memory_space=pl.ANY)],
        out_specs=pl.BlockSpec(memory_space=pltpu.VMEM),
        scratch_shapes=[
            pltpu.VMEM((m2, n), jnp.float32),
            pltpu.VMEM((m2, n), jnp.bfloat16),
            pltpu.VMEM((n_slots, hh, n), jnp.bfloat16),
            pltpu.SemaphoreType.DMA,
            pltpu.SemaphoreType.DMA((n_slots,)),
            pltpu.SemaphoreType.DMA((n_slots,)),
            pltpu.SemaphoreType.DMA((n_slots,)),
            pltpu.SemaphoreType.DMA((n_slots,)),
            pltpu.SemaphoreType.DMA((n_slots,)),
            pltpu.SemaphoreType.DMA((n_slots,)),
            pltpu.SemaphoreType.DMA((n_xg,)),
            pltpu.SemaphoreType.DMA((n_xg,)),
        ],
        compiler_params=pltpu.CompilerParams(collective_id=0),
    )(x)


# device time: 34806 ns/iter; 1.1680x vs baseline; 1.0043x over previous
import jax
import jax.numpy as jnp
from jax import lax
from jax.experimental import pallas as pl
from jax.experimental.pallas import tpu as pltpu

NZ = 4
S = 2


def kernel(x):
    m, n = x.shape
    m2 = m // 2
    ch = m2 // NZ
    hh = ch // S

    def body(x_hbm, out_ref, xv, xh, rs_buf, copy_sem, rs_send, rs_recv,
             ag_send, ag_recv, yf_send, yf_recv, xg_send, xg_recv):
        my_x = lax.axis_index("x")
        my_y = lax.axis_index("y")
        my_z = lax.axis_index("z")
        base = my_x * m2
        own = base + my_z * ch
        buddy_y = my_y + 1 - 2 * (my_y % 2)
        par = my_y % 2

        cp = pltpu.make_async_copy(
            x_hbm.at[pl.ds(base, m2), :], xv, copy_sem
        )
        cp.start()

        bsem = pltpu.get_barrier_semaphore()
        for k in range(1, NZ):
            pl.semaphore_signal(
                bsem, inc=1,
                device_id=(my_x, my_y, (my_z + k) % NZ),
                device_id_type=pl.DeviceIdType.MESH,
            )
        pl.semaphore_signal(
            bsem, inc=1,
            device_id=(1 - my_x, my_y, my_z),
            device_id_type=pl.DeviceIdType.MESH,
        )
        pl.semaphore_signal(
            bsem, inc=1,
            device_id=(my_x, buddy_y, my_z),
            device_id_type=pl.DeviceIdType.MESH,
        )
        cp.wait()
        xh[:, :] = xv[:, :].astype(jnp.bfloat16)
        pl.semaphore_wait(bsem, NZ + 1)

        rs = []
        for s in range(S):
            for k in range(1, NZ):
                dst = (my_z + k) % NZ
                slot = s * (NZ - 1) + (k - 1)
                rdma = pltpu.make_async_remote_copy(
                    src_ref=xh.at[pl.ds(dst * ch + s * hh, hh), :],
                    dst_ref=rs_buf.at[slot],
                    send_sem=rs_send.at[slot],
                    recv_sem=rs_recv.at[slot],
                    device_id=(my_x, my_y, dst),
                    device_id_type=pl.DeviceIdType.MESH,
                )
                rdma.start()
                rs.append(rdma)

        ag = []
        xg = []
        for s in range(S):
            for k in range(1, NZ):
                rs[s * (NZ - 1) + (k - 1)].wait_recv()
            acc = xh[pl.ds(my_z * ch + s * hh, hh), :]
            for k in range(1, NZ):
                slot = s * (NZ - 1) + (k - 1)
                acc = acc + rs_buf[slot, :, :]
            rows = pl.ds(own + s * hh, hh)
            out_ref[rows, :] = acc

            g = pltpu.make_async_remote_copy(
                src_ref=out_ref.at[rows, :],
                dst_ref=out_ref.at[rows, :],
                send_sem=xg_send.at[s],
                recv_sem=xg_recv.at[s],
                device_id=(1 - my_x, my_y, my_z),
                device_id_type=pl.DeviceIdType.MESH,
            )
            g.start()
            xg.append(g)

            for k in range(1, NZ):
                dst = (my_z + k) % NZ
                slot = s * (NZ - 1) + (k - 1)
                rdma = pltpu.make_async_remote_copy(
                    src_ref=out_ref.at[rows, :],
                    dst_ref=out_ref.at[rows, :],
                    send_sem=ag_send.at[slot],
                    recv_sem=ag_recv.at[slot],
                    device_id=(my_x, my_y, dst),
                    device_id_type=pl.DeviceIdType.MESH,
                )
                ag.append(rdma)

                @pl.when(my_z % 2 == par)
                def _(rdma=rdma):
                    rdma.start()

        yf = []
        for s in range(S):
            for k in range(1, NZ):
                slot = s * (NZ - 1) + (k - 1)
                c = (my_z - k) % NZ
                rows = pl.ds(base + c * ch + s * hh, hh)
                in_par = c % 2 == par

                @pl.when(in_par)
                def _(slot=slot):
                    ag[slot].wait_recv()

                gx = pltpu.make_async_remote_copy(
                    src_ref=out_ref.at[rows, :],
                    dst_ref=out_ref.at[rows, :],
                    send_sem=xg_send.at[S + slot],
                    recv_sem=xg_recv.at[S + slot],
                    device_id=(1 - my_x, my_y, my_z),
                    device_id_type=pl.DeviceIdType.MESH,
                )
                gy = pltpu.make_async_remote_copy(
                    src_ref=out_ref.at[rows, :],
                    dst_ref=out_ref.at[rows, :],
                    send_sem=yf_send.at[slot],
                    recv_sem=yf_recv.at[slot],
                    device_id=(my_x, buddy_y, my_z),
                    device_id_type=pl.DeviceIdType.MESH,
                )
                xg.append(gx)
                yf.append(gy)

                @pl.when(in_par)
                def _(gx=gx, gy=gy):
                    gx.start()
                    gy.start()

        for s in range(S):
            for k in range(1, NZ):
                slot = s * (NZ - 1) + (k - 1)
                c = (my_z - k) % NZ
                rows = pl.ds(base + c * ch + s * hh, hh)
                off_par = c % 2 != par

                @pl.when(off_par)
                def _(slot=slot):
                    yf[slot].wait_recv()

                gx = xg[S + slot]

                @pl.when(off_par)
                def _(gx=gx):
                    gx.start()

        for g in xg:
            g.wait_recv()
        for g in xg[:S]:
            g.wait_send()
        for s in range(S):
            for k in range(1, NZ):
                slot = s * (NZ - 1) + (k - 1)
                c = (my_z - k) % NZ

                @pl.when(c % 2 == par)
                def _(slot=slot):
                    xg[S + slot].wait_send()
                    yf[slot].wait_send()

                @pl.when(c % 2 != par)
                def _(slot=slot):
                    xg[S + slot].wait_send()

                @pl.when(my_z % 2 == par)
                def _(slot=slot):
                    ag[slot].wait_send()
        for rdma in rs:
            rdma.wait_send()

    n_slots = S * (NZ - 1)
    n_xg = S + n_slots

    return pl.pallas_call(
        body,
        out_shape=jax.ShapeDtypeStruct((m, n), jnp.bfloat16),
        in_specs=[pl.BlockSpec(memory_space=pl.ANY)],
        out_specs=pl.BlockSpec(memory_space=pltpu.VMEM),
        scratch_shapes=[
            pltpu.VMEM((m2, n), jnp.float32),
            pltpu.VMEM((m2, n), jnp.bfloat16),
            pltpu.VMEM((n_slots, hh, n), jnp.bfloat16),
            pltpu.SemaphoreType.DMA,
            pltpu.SemaphoreType.DMA((n_slots,)),
            pltpu.SemaphoreType.DMA((n_slots,)),
            pltpu.SemaphoreType.DMA((n_slots,)),
            pltpu.SemaphoreType.DMA((n_slots,)),
            pltpu.SemaphoreType.DMA((n_slots,)),
            pltpu.SemaphoreType.DMA((n_slots,)),
            pltpu.SemaphoreType.DMA((n_xg,)),
            pltpu.SemaphoreType.DMA((n_xg,)),
        ],
        compiler_params=pltpu.CompilerParams(collective_id=0),
    )(x)
